# baseline (speedup 1.0000x reference)
import functools

import jax
import jax.numpy as jnp
from jax import lax
from jax.experimental import pallas as pl
from jax.experimental.pallas import tpu as pltpu
from jax.experimental.pallas import tpu_sc as plsc


def _reduce_body(x_ref, idx_ref, maskf_ref):
    H = x_ref.shape[1]
    W = x_ref.shape[2]
    xb = x_ref[...].astype(jnp.bfloat16)
    sel = lax.broadcasted_iota(jnp.int32, (2, H), 0)
    hval = lax.broadcasted_iota(jnp.int32, (2, H), 1).astype(jnp.float32)
    lhs = jnp.where(sel == 0, jnp.float32(1), hval).astype(jnp.bfloat16)
    r = lax.dot_general(
        lhs, xb,
        dimension_numbers=(((1,), (1,)), ((), ())),
        preferred_element_type=jnp.float32,
    )
    colsum = r[0]
    hcolsum = r[1]
    wv = lax.broadcasted_iota(jnp.int32, colsum.shape, 1).astype(jnp.float32)
    count = jnp.sum(colsum, axis=1)
    xsum = jnp.sum(colsum * wv, axis=1)
    ysum = jnp.sum(hcolsum, axis=1)
    safe = jnp.maximum(count, 1.0)
    nz = count > 0.0
    ty = jnp.where(nz, ysum / safe, 0.0).astype(jnp.int32)
    tx = jnp.where(nz, xsum / safe, 0.0).astype(jnp.int32)
    idx_ref[...] = ty * W + tx
    maskf_ref[...] = nz.astype(jnp.float32)[:, None]


def _tc_reduce(x):
    B, H, W = x.shape
    BB = 128
    grid = B // BB
    return pl.pallas_call(
        _reduce_body,
        grid=(grid,),
        in_specs=[pl.BlockSpec((BB, H, W), lambda i: (i, 0, 0))],
        out_specs=[
            pl.BlockSpec((BB,), lambda i: (i,)),
            pl.BlockSpec((BB, 1), lambda i: (i, 0)),
        ],
        out_shape=[
            jax.ShapeDtypeStruct((B,), jnp.int32),
            jax.ShapeDtypeStruct((B, 1), jnp.float32),
        ],
    )(x)


def _make_sc_gather(B, D):
    info = plsc.get_sparse_core_info()
    NC, NS = info.num_cores, info.num_subcores
    NW = NC * NS
    assert B % (8 * NW) == 0
    bpw = B // NW
    mesh = plsc.VectorSubcoreMesh(core_axis_name="c", subcore_axis_name="s")

    @functools.partial(
        pl.kernel,
        mesh=mesh,
        out_type=jax.ShapeDtypeStruct((B, D), jnp.float32),
        scratch_types=[
            pltpu.VMEM((bpw,), jnp.int32),
            pltpu.VMEM((bpw, D), jnp.float32),
            pltpu.SemaphoreType.DMA,
        ],
    )
    def sc_k(table_hbm, idx_hbm, out_hbm, idx_v, rows_v, sem):
        wid = lax.axis_index("s") * NC + lax.axis_index("c")
        base = wid * bpw
        pltpu.sync_copy(idx_hbm.at[pl.ds(base, bpw)], idx_v)
        pltpu.async_copy(table_hbm.at[idx_v], rows_v, sem).wait()
        pltpu.sync_copy(rows_v, out_hbm.at[pl.ds(base, bpw)])

    return sc_k


def _combine_body(rows_ref, maskf_ref, g_ref, e_ref, out_ref):
    rows = rows_ref[...]
    m = maskf_ref[...]
    gme = (g_ref[...] - e_ref[...])[None, :]
    out_ref[...] = e_ref[...][None, :] + m * (rows + gme)


def _tc_combine(rows, maskf, g, e):
    B, D = rows.shape
    BB = 256
    grid = B // BB
    return pl.pallas_call(
        _combine_body,
        grid=(grid,),
        in_specs=[
            pl.BlockSpec((BB, D), lambda i: (i, 0)),
            pl.BlockSpec((BB, 1), lambda i: (i, 0)),
            pl.BlockSpec((D,), lambda i: (0,)),
            pl.BlockSpec((D,), lambda i: (0,)),
        ],
        out_specs=pl.BlockSpec((BB, D), lambda i: (i, 0)),
        out_shape=jax.ShapeDtypeStruct((B, D), jnp.float32),
    )(rows, maskf, g, e)


def kernel(x, pe, global_emb, empty_emb):
    B, H, W = x.shape
    D = pe.shape[-1]
    x8 = x.view(jnp.int8)
    idx, maskf = _tc_reduce(x8)
    table = pe.reshape(H * W, D)
    sc_k = _make_sc_gather(B, D)
    rows = sc_k(table, idx)
    return _tc_combine(rows, maskf, global_emb, empty_emb)
